# baseline (speedup 1.0000x reference)
import jax
import jax.numpy as jnp
from jax.experimental import pallas as pl

B, N, T, P, D = 256, 128, 1, 7, 256
TOKEN_DIM = 128
EMB_FEAT = P * D
M = B * N

ROWS = 512


def _fused_masked_linear(x_ref, vis_ref, mask_ref, w1_ref, w2_ref, b_ref, out_ref):
    acc = jnp.dot(x_ref[:], w1_ref[:], preferred_element_type=jnp.float32)
    acc += jnp.dot(vis_ref[:], w2_ref[:], preferred_element_type=jnp.float32)
    acc += b_ref[:]
    out_ref[:] = acc * mask_ref[:]


def kernel(embs, vis, masks, W, b):
    x2d = embs.reshape(M, EMB_FEAT)
    vis2d = vis.reshape(M, P)
    maskf = masks.reshape(M, 1).astype(jnp.float32)
    w1 = W[:, :EMB_FEAT].T
    w2 = W[:, EMB_FEAT:].T
    b2 = b.reshape(1, TOKEN_DIM)

    grid = (M // ROWS,)
    out = pl.pallas_call(
        _fused_masked_linear,
        grid=grid,
        in_specs=[
            pl.BlockSpec((ROWS, EMB_FEAT), lambda i: (i, 0)),
            pl.BlockSpec((ROWS, P), lambda i: (i, 0)),
            pl.BlockSpec((ROWS, 1), lambda i: (i, 0)),
            pl.BlockSpec((EMB_FEAT, TOKEN_DIM), lambda i: (0, 0)),
            pl.BlockSpec((P, TOKEN_DIM), lambda i: (0, 0)),
            pl.BlockSpec((1, TOKEN_DIM), lambda i: (0, 0)),
        ],
        out_specs=pl.BlockSpec((ROWS, TOKEN_DIM), lambda i: (i, 0)),
        out_shape=jax.ShapeDtypeStruct((M, TOKEN_DIM), jnp.float32),
    )(x2d, vis2d, maskf, w1, w2, b2)
    return out.reshape(B, N, TOKEN_DIM)


# speedup vs baseline: 1.0024x; 1.0024x over previous
import jax
import jax.numpy as jnp
from jax.experimental import pallas as pl

B, N, T, P, D = 256, 128, 1, 7, 256
TOKEN_DIM = 128
EMB_FEAT = P * D
M = B * N

ROWS = 512


def _fused_masked_linear(x_ref, vis_ref, mask_ref, w1_ref, w2_ref, b_ref, out_ref):
    x = x_ref[:].astype(jnp.bfloat16)
    acc = jnp.dot(x, w1_ref[:], preferred_element_type=jnp.float32)
    acc += jnp.dot(vis_ref[:], w2_ref[:], preferred_element_type=jnp.float32)
    acc += b_ref[:]
    out_ref[:] = acc * mask_ref[:]


def kernel(embs, vis, masks, W, b):
    x2d = embs.reshape(M, EMB_FEAT)
    vis2d = vis.reshape(M, P)
    maskf = masks.reshape(M, 1).astype(jnp.float32)
    w1 = W[:, :EMB_FEAT].T.astype(jnp.bfloat16)
    w2 = W[:, EMB_FEAT:].T
    b2 = b.reshape(1, TOKEN_DIM)

    grid = (M // ROWS,)
    out = pl.pallas_call(
        _fused_masked_linear,
        grid=grid,
        in_specs=[
            pl.BlockSpec((ROWS, EMB_FEAT), lambda i: (i, 0)),
            pl.BlockSpec((ROWS, P), lambda i: (i, 0)),
            pl.BlockSpec((ROWS, 1), lambda i: (i, 0)),
            pl.BlockSpec((EMB_FEAT, TOKEN_DIM), lambda i: (0, 0)),
            pl.BlockSpec((P, TOKEN_DIM), lambda i: (0, 0)),
            pl.BlockSpec((1, TOKEN_DIM), lambda i: (0, 0)),
        ],
        out_specs=pl.BlockSpec((ROWS, TOKEN_DIM), lambda i: (i, 0)),
        out_shape=jax.ShapeDtypeStruct((M, TOKEN_DIM), jnp.float32),
    )(x2d, vis2d, maskf, w1, w2, b2)
    return out.reshape(B, N, TOKEN_DIM)


# speedup vs baseline: 2.0260x; 2.0210x over previous
import jax
import jax.numpy as jnp
from jax.experimental import pallas as pl

B, N, T, P, D = 256, 128, 1, 7, 256
TOKEN_DIM = 128
EMB_FEAT = P * D
M = B * N

ROWS = 512


def _fused_masked_linear(x_ref, vis_ref, mask_ref, w1_ref, w2_ref, b_ref, out_ref):
    acc = jnp.dot(vis_ref[:], w2_ref[:], preferred_element_type=jnp.float32)
    acc += b_ref[:]
    for p in range(P):
        x = x_ref[:, p, :].astype(jnp.bfloat16)
        acc += jnp.dot(x, w1_ref[p], preferred_element_type=jnp.float32)
    out_ref[:] = acc * mask_ref[:]


def kernel(embs, vis, masks, W, b):
    x3d = embs.reshape(M, P, D)
    vis2d = vis.reshape(M, P)
    maskf = masks.reshape(M, 1).astype(jnp.float32)
    w1 = W[:, :EMB_FEAT].T.reshape(P, D, TOKEN_DIM).astype(jnp.bfloat16)
    w2 = W[:, EMB_FEAT:].T
    b2 = b.reshape(1, TOKEN_DIM)

    grid = (M // ROWS,)
    out = pl.pallas_call(
        _fused_masked_linear,
        grid=grid,
        in_specs=[
            pl.BlockSpec((ROWS, P, D), lambda i: (i, 0, 0)),
            pl.BlockSpec((ROWS, P), lambda i: (i, 0)),
            pl.BlockSpec((ROWS, 1), lambda i: (i, 0)),
            pl.BlockSpec((P, D, TOKEN_DIM), lambda i: (0, 0, 0)),
            pl.BlockSpec((P, TOKEN_DIM), lambda i: (0, 0)),
            pl.BlockSpec((1, TOKEN_DIM), lambda i: (0, 0)),
        ],
        out_specs=pl.BlockSpec((ROWS, TOKEN_DIM), lambda i: (i, 0)),
        out_shape=jax.ShapeDtypeStruct((M, TOKEN_DIM), jnp.float32),
    )(x3d, vis2d, maskf, w1, w2, b2)
    return out.reshape(B, N, TOKEN_DIM)
